# baseline (speedup 1.0000x reference)
import jax
import jax.numpy as jnp
from jax.experimental import pallas as pl
from jax.experimental.pallas import tpu as pltpu

B, N = 4096, 50
EMB, VIS, K, TOK = 128, 1, 17, 128
KF = K * 3
BB = 64


def _fused(emb_ref, kpt_ref, bbox_ref, vis_ref, mask_ref,
           wemb_ref, wkpt_ref, wbbox_ref, wvis_ref, bias_ref, out_ref):
    wemb = wemb_ref[:]
    wkpt = wkpt_ref[:]
    wbbox = wbbox_ref[:]
    wvis = wvis_ref[:]
    bias = bias_ref[:]
    for n in range(N):
        acc = jnp.dot(emb_ref[:, n, :], wemb,
                      preferred_element_type=jnp.float32)
        acc += jnp.dot(kpt_ref[:, n * KF:(n + 1) * KF], wkpt,
                       preferred_element_type=jnp.float32)
        acc += jnp.dot(bbox_ref[:, n * 4:(n + 1) * 4], wbbox,
                       preferred_element_type=jnp.float32)
        acc += vis_ref[:, n:n + 1] * wvis
        acc += bias
        out_ref[:, n, :] = acc * mask_ref[:, n:n + 1]


def kernel(embeddings, visibility_scores, bbox_ltwh, keypoints_xyc,
           W_app, b_app, W_st, b_st, feats_masks):
    kpt2 = keypoints_xyc.reshape(B, N * KF)
    bbox2 = bbox_ltwh.reshape(B, N * 4)
    vis2 = visibility_scores.reshape(B, N)
    maskf = feats_masks.astype(jnp.float32)

    w_emb = W_app[:, :EMB].T
    w_vis = W_app[:, EMB:].T
    w_bbox = W_st[:, :4].T
    w_kpt = W_st[:, 4:].T
    bias = (b_app + b_st).reshape(1, TOK)

    out = pl.pallas_call(
        _fused,
        grid=(B // BB,),
        in_specs=[
            pl.BlockSpec((BB, N, EMB), lambda i: (i, 0, 0)),
            pl.BlockSpec((BB, N * KF), lambda i: (i, 0)),
            pl.BlockSpec((BB, N * 4), lambda i: (i, 0)),
            pl.BlockSpec((BB, N), lambda i: (i, 0)),
            pl.BlockSpec((BB, N), lambda i: (i, 0)),
            pl.BlockSpec((EMB, TOK), lambda i: (0, 0)),
            pl.BlockSpec((KF, TOK), lambda i: (0, 0)),
            pl.BlockSpec((4, TOK), lambda i: (0, 0)),
            pl.BlockSpec((VIS, TOK), lambda i: (0, 0)),
            pl.BlockSpec((1, TOK), lambda i: (0, 0)),
        ],
        out_specs=pl.BlockSpec((BB, N, TOK), lambda i: (i, 0, 0)),
        out_shape=jax.ShapeDtypeStruct((B, N, TOK), jnp.float32),
        compiler_params=pltpu.CompilerParams(
            dimension_semantics=("parallel",)),
    )(embeddings, kpt2, bbox2, vis2, maskf,
      w_emb, w_kpt, w_bbox, w_vis, bias)

    return out
